# Initial kernel scaffold; baseline (speedup 1.0000x reference)
import jax
import jax.numpy as jnp
from jax.experimental import pallas as pl


def kernel(hidden_states, gate_w, W1, b1, W2, b2, shared_W1, shared_b1, shared_W2, shared_b2, sgate_w, sgate_b):
    raise NotImplementedError("write your pallas kernel here")


# speedup vs baseline: 1.2229x; 1.2229x over previous
import functools

import jax
import jax.numpy as jnp
from jax.experimental import pallas as pl

NUM_EXPERTS = 8
TOP_K = 2
D_MODEL = 1024
D_FF = 512
T_TOKENS = 2048
TB = 256


def _dot_bf16(a, b):
    return jax.lax.dot_general(
        a, b, (((1,), (0,)), ((), ())), preferred_element_type=jnp.float32
    )


def _moe_block_kernel(x_ref, gate_ref, w1_ref, b1_ref, w2_ref, b2_ref,
                      sw1_ref, sb1_ref, sw2_ref, sb2_ref, sgw_ref, sgb_ref,
                      out_ref):
    x = x_ref[...]
    x_bf = x.astype(jnp.bfloat16)

    logits = _dot_bf16(x_bf, gate_ref[...].astype(jnp.bfloat16))
    m = jnp.max(logits, axis=-1, keepdims=True)
    e = jnp.exp(logits - m)
    probs = e / jnp.sum(e, axis=-1, keepdims=True)

    iota = jax.lax.broadcasted_iota(jnp.int32, probs.shape, 1)
    w1 = jnp.max(probs, axis=-1, keepdims=True)
    is_max = probs == w1
    i1 = jnp.min(jnp.where(is_max, iota, NUM_EXPERTS), axis=-1, keepdims=True)
    mask1 = iota == i1
    probs2 = jnp.where(mask1, -jnp.inf, probs)
    w2 = jnp.max(probs2, axis=-1, keepdims=True)
    is_max2 = probs2 == w2
    i2 = jnp.min(jnp.where(is_max2, iota, NUM_EXPERTS), axis=-1, keepdims=True)
    mask2 = iota == i2
    denom = w1 + w2
    combine = jnp.where(mask1 | mask2, probs, 0.0) / denom

    acc = jnp.zeros((TB, D_MODEL), jnp.float32)
    for ex in range(NUM_EXPERTS):
        h = _dot_bf16(x_bf, w1_ref[ex]) + b1_ref[ex][None, :]
        h = jax.nn.gelu(h)
        y = _dot_bf16(h.astype(jnp.bfloat16), w2_ref[ex]) + b2_ref[ex][None, :]
        acc = acc + combine[:, ex:ex + 1] * y

    hs = _dot_bf16(x_bf, sw1_ref[...]) + sb1_ref[...]
    hs = jax.nn.gelu(hs)
    ys = _dot_bf16(hs.astype(jnp.bfloat16), sw2_ref[...]) + sb2_ref[...]
    glog = _dot_bf16(x_bf, sgw_ref[...].astype(jnp.bfloat16)) + sgb_ref[...]
    g = jax.nn.sigmoid(glog)
    out_ref[...] = acc + g * ys


@jax.jit
def kernel(hidden_states, gate_w, W1, b1, W2, b2, shared_W1, shared_b1,
           shared_W2, shared_b2, sgate_w, sgate_b):
    T, D = hidden_states.shape
    num_blocks = T // TB

    w1_bf = W1.astype(jnp.bfloat16)
    w2_bf = W2.astype(jnp.bfloat16)
    sw1_bf = shared_W1.astype(jnp.bfloat16)
    sw2_bf = shared_W2.astype(jnp.bfloat16)
    sb1_2d = shared_b1.reshape(1, D_FF)
    sb2_2d = shared_b2.reshape(1, D_MODEL)
    sgb_2d = sgate_b.reshape(1, 1)

    full = lambda *shape: pl.BlockSpec(shape, lambda i: (0,) * len(shape))
    out = pl.pallas_call(
        _moe_block_kernel,
        grid=(num_blocks,),
        in_specs=[
            pl.BlockSpec((TB, D), lambda i: (i, 0)),
            full(D, NUM_EXPERTS),
            full(NUM_EXPERTS, D, D_FF),
            full(NUM_EXPERTS, D_FF),
            full(NUM_EXPERTS, D_FF, D),
            full(NUM_EXPERTS, D),
            full(D, D_FF),
            full(1, D_FF),
            full(D_FF, D),
            full(1, D),
            full(D, 1),
            full(1, 1),
        ],
        out_specs=pl.BlockSpec((TB, D), lambda i: (i, 0)),
        out_shape=jax.ShapeDtypeStruct((T, D), jnp.float32),
    )(hidden_states, gate_w, w1_bf, b1, w2_bf, b2, sw1_bf, sb1_2d,
      sw2_bf, sb2_2d, sgate_w, sgb_2d)
    return out


# speedup vs baseline: 1.5011x; 1.2274x over previous
import jax
import jax.numpy as jnp
from jax.experimental import pallas as pl

NUM_EXPERTS = 8
TOP_K = 2
D_MODEL = 1024
D_FF = 512
TB = 256


def _dot(a, b):
    return jax.lax.dot_general(
        a, b, (((1,), (0,)), ((), ())), preferred_element_type=jnp.float32
    )


def _moe_block_kernel(x_ref, gate_ref, w1_ref, b1_ref, w2_ref, b2_ref,
                      sw1_ref, sb1_ref, sw2_ref, sb2_ref, sgw_ref, sgb_ref,
                      out_ref):
    x = x_ref[...]

    logits = _dot(x, gate_ref[...])
    m = jnp.max(logits, axis=-1, keepdims=True)
    e = jnp.exp(logits - m)
    probs = e / jnp.sum(e, axis=-1, keepdims=True)

    iota = jax.lax.broadcasted_iota(jnp.int32, probs.shape, 1)
    w1 = jnp.max(probs, axis=-1, keepdims=True)
    is_max = probs == w1
    i1 = jnp.min(jnp.where(is_max, iota, NUM_EXPERTS), axis=-1, keepdims=True)
    mask1 = iota == i1
    probs2 = jnp.where(mask1, -jnp.inf, probs)
    w2 = jnp.max(probs2, axis=-1, keepdims=True)
    is_max2 = probs2 == w2
    i2 = jnp.min(jnp.where(is_max2, iota, NUM_EXPERTS), axis=-1, keepdims=True)
    mask2 = iota == i2
    denom = w1 + w2
    combine = jnp.where(mask1 | mask2, probs, 0.0) / denom

    acc = jnp.zeros((TB, D_MODEL), jnp.float32)
    for ex in range(NUM_EXPERTS):
        h = _dot(x, w1_ref[ex]) + b1_ref[ex][None, :]
        h = jax.nn.gelu(h)
        y = _dot(h, w2_ref[ex]) + b2_ref[ex][None, :]
        acc = acc + combine[:, ex:ex + 1] * y

    hs = _dot(x, sw1_ref[...]) + sb1_ref[...]
    hs = jax.nn.gelu(hs)
    ys = _dot(hs, sw2_ref[...]) + sb2_ref[...]
    glog = _dot(x, sgw_ref[...]) + sgb_ref[...]
    g = jax.nn.sigmoid(glog)
    out_ref[...] = acc + g * ys


@jax.jit
def kernel(hidden_states, gate_w, W1, b1, W2, b2, shared_W1, shared_b1,
           shared_W2, shared_b2, sgate_w, sgate_b):
    T, D = hidden_states.shape
    num_blocks = T // TB

    sb1_2d = shared_b1.reshape(1, D_FF)
    sb2_2d = shared_b2.reshape(1, D_MODEL)
    sgb_2d = sgate_b.reshape(1, 1)

    full = lambda *shape: pl.BlockSpec(shape, lambda i: (0,) * len(shape))
    out = pl.pallas_call(
        _moe_block_kernel,
        grid=(num_blocks,),
        in_specs=[
            pl.BlockSpec((TB, D), lambda i: (i, 0)),
            full(D, NUM_EXPERTS),
            full(NUM_EXPERTS, D, D_FF),
            full(NUM_EXPERTS, D_FF),
            full(NUM_EXPERTS, D_FF, D),
            full(NUM_EXPERTS, D),
            full(D, D_FF),
            full(1, D_FF),
            full(D_FF, D),
            full(1, D),
            full(D, 1),
            full(1, 1),
        ],
        out_specs=pl.BlockSpec((TB, D), lambda i: (i, 0)),
        out_shape=jax.ShapeDtypeStruct((T, D), jnp.float32),
    )(hidden_states, gate_w, W1, b1, W2, b2, shared_W1, sb1_2d,
      shared_W2, sb2_2d, sgate_w, sgb_2d)
    return out


# speedup vs baseline: 1.7368x; 1.1570x over previous
import jax
import jax.numpy as jnp
from jax.experimental import pallas as pl

NUM_EXPERTS = 8
TOP_K = 2
D_MODEL = 1024
D_FF = 512
TB = 512


def _dot(a, b):
    return jax.lax.dot_general(
        a, b, (((1,), (0,)), ((), ())), preferred_element_type=jnp.float32
    )


def _moe_block_kernel(x_ref, gate_ref, w1_ref, b1_ref, w2_ref, b2_ref,
                      sw1_ref, sb1_ref, sw2_ref, sb2_ref, sgw_ref, sgb_ref,
                      out_ref):
    x = x_ref[...]

    logits = _dot(x, gate_ref[...])
    m = jnp.max(logits, axis=-1, keepdims=True)
    e = jnp.exp(logits - m)
    probs = e / jnp.sum(e, axis=-1, keepdims=True)

    iota = jax.lax.broadcasted_iota(jnp.int32, probs.shape, 1)
    w1 = jnp.max(probs, axis=-1, keepdims=True)
    is_max = probs == w1
    i1 = jnp.min(jnp.where(is_max, iota, NUM_EXPERTS), axis=-1, keepdims=True)
    mask1 = iota == i1
    probs2 = jnp.where(mask1, -jnp.inf, probs)
    w2 = jnp.max(probs2, axis=-1, keepdims=True)
    is_max2 = probs2 == w2
    i2 = jnp.min(jnp.where(is_max2, iota, NUM_EXPERTS), axis=-1, keepdims=True)
    mask2 = iota == i2
    denom = w1 + w2
    combine = jnp.where(mask1 | mask2, probs, 0.0) / denom

    acc = jnp.zeros((TB, D_MODEL), jnp.float32)
    for ex in range(NUM_EXPERTS):
        h = _dot(x, w1_ref[ex]) + b1_ref[ex][None, :]
        h = jax.nn.gelu(h)
        y = _dot(h, w2_ref[ex]) + b2_ref[ex][None, :]
        acc = acc + combine[:, ex:ex + 1] * y

    hs = _dot(x, sw1_ref[...]) + sb1_ref[...]
    hs = jax.nn.gelu(hs)
    ys = _dot(hs, sw2_ref[...]) + sb2_ref[...]
    glog = _dot(x, sgw_ref[...]) + sgb_ref[...]
    g = jax.nn.sigmoid(glog)
    out_ref[...] = acc + g * ys


@jax.jit
def kernel(hidden_states, gate_w, W1, b1, W2, b2, shared_W1, shared_b1,
           shared_W2, shared_b2, sgate_w, sgate_b):
    T, D = hidden_states.shape
    num_blocks = T // TB

    sb1_2d = shared_b1.reshape(1, D_FF)
    sb2_2d = shared_b2.reshape(1, D_MODEL)
    sgb_2d = sgate_b.reshape(1, 1)

    full = lambda *shape: pl.BlockSpec(shape, lambda i: (0,) * len(shape))
    out = pl.pallas_call(
        _moe_block_kernel,
        grid=(num_blocks,),
        in_specs=[
            pl.BlockSpec((TB, D), lambda i: (i, 0)),
            full(D, NUM_EXPERTS),
            full(NUM_EXPERTS, D, D_FF),
            full(NUM_EXPERTS, D_FF),
            full(NUM_EXPERTS, D_FF, D),
            full(NUM_EXPERTS, D),
            full(D, D_FF),
            full(1, D_FF),
            full(D_FF, D),
            full(1, D),
            full(D, 1),
            full(1, 1),
        ],
        out_specs=pl.BlockSpec((TB, D), lambda i: (i, 0)),
        out_shape=jax.ShapeDtypeStruct((T, D), jnp.float32),
    )(hidden_states, gate_w, W1, b1, W2, b2, shared_W1, sb1_2d,
      shared_W2, sb2_2d, sgate_w, sgb_2d)
    return out
